# Initial kernel scaffold; baseline (speedup 1.0000x reference)
import jax
import jax.numpy as jnp
from jax.experimental import pallas as pl


def kernel(x, fc_w, fc_b, proj_w, proj_b, route_w, route_b):
    raise NotImplementedError("write your pallas kernel here")


# speedup vs baseline: 7.8871x; 7.8871x over previous
import functools
import math

import jax
import jax.numpy as jnp
from jax.experimental import pallas as pl
from jax.experimental.pallas import tpu as pltpu


def _gelu(x):
    theta_x = 1 + jnp.tanh(math.sqrt(2 / math.pi) * (x + 0.044715 * jnp.power(x, 3)))
    return 0.5 * x * theta_x


def _routing_kernel(x_ref, rw_ref, rb_ref, vals_ref, idx_ref, acc_ref, *, n_steps, n_total):
    step = pl.program_id(0)
    part = jnp.sum(x_ref[...], axis=0, keepdims=True)

    @pl.when(step == 0)
    def _init():
        acc_ref[...] = part

    @pl.when(step > 0)
    def _acc():
        acc_ref[...] += part

    @pl.when(step == n_steps - 1)
    def _finish():
        mean_x = acc_ref[...] / n_total
        scores = jnp.dot(mean_x, rw_ref[...],
                         preferred_element_type=jnp.float32) + rb_ref[...]
        m = jnp.max(scores, axis=1, keepdims=True)
        e = jnp.exp(scores - m)
        p = e / jnp.sum(e, axis=1, keepdims=True)
        i0 = jnp.argmax(p, axis=1)[0]
        v0 = jnp.max(p, axis=1)[0]
        col = jax.lax.broadcasted_iota(jnp.int32, p.shape, 1)
        p2 = jnp.where(col == i0, -jnp.inf, p)
        i1 = jnp.argmax(p2, axis=1)[0]
        v1 = jnp.max(p2, axis=1)[0]
        vals_ref[0] = v0
        vals_ref[1] = v1
        idx_ref[0] = i0.astype(jnp.int32)
        idx_ref[1] = i1.astype(jnp.int32)


def _ffn_kernel(idx_ref, xk_ref, fcw_ref, fcb_ref, pjw_ref, pjb_ref, vals_ref,
                out_ref, acc_ref, *, n_out):
    j = pl.program_id(0)
    xv = xk_ref[0]
    h = jnp.dot(xv, fcw_ref[0], preferred_element_type=jnp.float32)
    h = _gelu(h + fcb_ref[0])
    y = jnp.dot(h, pjw_ref[0], preferred_element_type=jnp.float32)
    y = y + pjb_ref[0]
    contrib = vals_ref[j] * y

    @pl.when(j == 0)
    def _init():
        acc_ref[...] = contrib

    @pl.when(j == 1)
    def _finish():
        row = acc_ref[...] + contrib
        out_ref[...] = jnp.broadcast_to(row, (n_out, row.shape[1]))


def kernel(x, fc_w, fc_b, proj_w, proj_b, route_w, route_b):
    B, N, D = x.shape
    E, _, F = fc_w.shape
    K = 2
    x2 = x[0]

    n_steps = 8
    tile = N // n_steps
    vals, idx = pl.pallas_call(
        functools.partial(_routing_kernel, n_steps=n_steps, n_total=float(N)),
        grid=(n_steps,),
        in_specs=[
            pl.BlockSpec((tile, D), lambda s: (s, 0)),
            pl.BlockSpec((D, E), lambda s: (0, 0)),
            pl.BlockSpec((1, E), lambda s: (0, 0)),
        ],
        out_specs=[
            pl.BlockSpec(memory_space=pltpu.SMEM),
            pl.BlockSpec(memory_space=pltpu.SMEM),
        ],
        out_shape=[
            jax.ShapeDtypeStruct((K,), jnp.float32),
            jax.ShapeDtypeStruct((K,), jnp.int32),
        ],
        scratch_shapes=[pltpu.VMEM((1, D), jnp.float32)],
    )(x2, route_w, route_b.reshape(1, E))

    out2 = pl.pallas_call(
        functools.partial(_ffn_kernel, n_out=N),
        grid_spec=pltpu.PrefetchScalarGridSpec(
            num_scalar_prefetch=1,
            grid=(K,),
            in_specs=[
                pl.BlockSpec((1, 1, D), lambda j, idx_ref: (j, 0, 0)),
                pl.BlockSpec((1, D, F), lambda j, idx_ref: (idx_ref[j], 0, 0)),
                pl.BlockSpec((1, 1, F), lambda j, idx_ref: (idx_ref[j], 0, 0)),
                pl.BlockSpec((1, F, D), lambda j, idx_ref: (idx_ref[j], 0, 0)),
                pl.BlockSpec((1, 1, D), lambda j, idx_ref: (idx_ref[j], 0, 0)),
                pl.BlockSpec(memory_space=pltpu.SMEM),
            ],
            out_specs=pl.BlockSpec((N, D), lambda j, idx_ref: (0, 0)),
            scratch_shapes=[pltpu.VMEM((1, D), jnp.float32)],
        ),
        out_shape=jax.ShapeDtypeStruct((N, D), jnp.float32),
    )(idx, x2[:K].reshape(K, 1, D), fc_w, fc_b.reshape(E, 1, F),
      proj_w, proj_b.reshape(E, 1, D), vals)

    return out2[None]
